# Initial kernel scaffold; baseline (speedup 1.0000x reference)
import functools

import jax
import jax.numpy as jnp
from jax import lax
from jax.experimental import pallas as pl
from jax.experimental.pallas import tpu as pltpu
from jax.experimental.pallas import tpu_sc as plsc

N = 10000
E = 320000
D_IN = 128
C = 64
NB = 4

NC = 2
NS = 16
NW = NC * NS

K = 80
EPW2 = 2 * E // NW
EPT = E // NS
NPT = N // NS
NPQ = 125

_MESH = plsc.VectorSubcoreMesh(core_axis_name="c", subcore_axis_name="s")


def _node_ff_body(p1_ref, wpp_ref, bpp_ref, h_ref):
    x = jnp.dot(p1_ref[...], wpp_ref[...], preferred_element_type=jnp.float32)
    h_ref[...] = jnp.maximum(x + bpp_ref[...], 0.0)


def _edge_ff_body(hcat_ref, basis_ref, wpi_ref, bpi_ref, wii_ref, out_ref):
    inter = jnp.dot(hcat_ref[...], wpi_ref[...],
                    preferred_element_type=jnp.float32) + bpi_ref[...]
    bs = basis_ref[...]
    i1 = (inter[:, 0:C] * bs[:, 0:1] + inter[:, C:2 * C] * bs[:, 1:2]
          + inter[:, 2 * C:3 * C] * bs[:, 2:3] + inter[:, 3 * C:4 * C] * bs[:, 3:4])
    t = jnp.dot(i1, wii_ref[...], preferred_element_type=jnp.float32)
    out_ref[...] = jnp.maximum(t, 0.0)


def _finalize_body(a_ref, b_ref, p1t_ref, p3t_ref):
    a = a_ref[...]
    b = b_ref[...]
    p1n = a[:, 0:C]
    p30 = a[:, C:2 * C]
    p31 = b[:, 0:C]
    p32 = b[:, C:2 * C]
    s = p30 * p30 + p31 * p31 + p32 * p32 + p1n
    p1t_ref[...] = s
    p3t_ref[:, 0, :] = p30 * s
    p3t_ref[:, 1, :] = p31 * s
    p3t_ref[:, 2, :] = p32 * s


@functools.partial(
    pl.kernel,
    out_type=jax.ShapeDtypeStruct((2 * E, C), jnp.float32),
    mesh=_MESH,
    scratch_types=[
        pltpu.VMEM((K,), jnp.int32),
        pltpu.VMEM((K, C), jnp.float32),
        pltpu.SemaphoreType.DMA,
    ],
)
def _sc_gather(h_hbm, idx_hbm, out_hbm, idx_v, rows_v, sem):
    wid = lax.axis_index("s") * NC + lax.axis_index("c")
    base_w = wid * EPW2

    def chunk(t, carry):
        base = base_w + t * K
        pltpu.sync_copy(idx_hbm.at[pl.ds(base, K)], idx_v)
        pltpu.async_copy(h_hbm.at[idx_v], rows_v, sem).wait()
        pltpu.sync_copy(rows_v, out_hbm.at[pl.ds(base, K)])
        return carry

    lax.fori_loop(0, EPW2 // K, chunk, 0)


@functools.partial(
    pl.kernel,
    out_type=jax.ShapeDtypeStruct((NC, N, 2 * C), jnp.float32),
    mesh=_MESH,
    scratch_types=[
        pltpu.VMEM((K,), jnp.int32),
        pltpu.VMEM((K,), jnp.int32),
        pltpu.VMEM((K, 3 * C), jnp.float32),
        pltpu.VMEM((K, C), jnp.float32),
        pltpu.VMEM((K, C), jnp.float32),
        pltpu.VMEM((K,), jnp.float32),
        pltpu.VMEM((K,), jnp.float32),
        pltpu.VMEM((K, 2 * C), jnp.float32),
        pltpu.VMEM((NPQ, 2 * C), jnp.float32),
        pltpu.VMEM_SHARED((N, 2 * C), jnp.float32),
        pltpu.SemaphoreType.DMA,
    ],
)
def _sc_scatter(i1_hbm, ii_hbm, jj_hbm, d0_hbm, d1_hbm, d2_hbm,
                t0_hbm, t1_hbm, t2_hbm, out_hbm,
                idxd_v, idxs_v, i1_v, pa_v, pb_v, da_v, db_v, row_v,
                cb_v, acc, sem):
    c = lax.axis_index("c")
    s = lax.axis_index("s")

    zero16 = jnp.zeros((16,), jnp.float32)

    def zrow(r, carry):
        for g in range(2 * C // 16):
            cb_v[r, pl.ds(g * 16, 16)] = zero16
        return carry

    lax.fori_loop(0, NPQ, zrow, 0)
    nbase = s * NPT
    for q in range(NPT // NPQ):
        pltpu.sync_copy(cb_v, acc.at[pl.ds(nbase + q * NPQ, NPQ)])
    plsc.subcore_barrier()

    ebase = s * EPT

    def chunk(t, carry):
        base = ebase + t * K
        pltpu.sync_copy(ii_hbm.at[pl.ds(base, K)], idxd_v)
        pltpu.sync_copy(jj_hbm.at[pl.ds(base, K)], idxs_v)
        pltpu.sync_copy(i1_hbm.at[pl.ds(base, K)], i1_v)

        @pl.when(c == 0)
        def _():
            pltpu.sync_copy(d0_hbm.at[pl.ds(base, K)], da_v)
            pltpu.async_copy(t0_hbm.at[idxs_v], pa_v, sem).wait()

            def ebody(e, cc):
                dd = da_v[e]
                for g in range(C // 16):
                    a1 = i1_v[e, pl.ds(g * 16, 16)]
                    a2 = i1_v[e, pl.ds(C + g * 16, 16)]
                    a3 = i1_v[e, pl.ds(2 * C + g * 16, 16)]
                    pj = pa_v[e, pl.ds(g * 16, 16)]
                    row_v[e, pl.ds(g * 16, 16)] = a2
                    row_v[e, pl.ds(C + g * 16, 16)] = pj * a3 + dd * a1
                return cc

            lax.fori_loop(0, K, ebody, 0)

        @pl.when(c == 1)
        def _():
            pltpu.sync_copy(d1_hbm.at[pl.ds(base, K)], da_v)
            pltpu.sync_copy(d2_hbm.at[pl.ds(base, K)], db_v)
            cp1 = pltpu.async_copy(t1_hbm.at[idxs_v], pa_v, sem)
            cp2 = pltpu.async_copy(t2_hbm.at[idxs_v], pb_v, sem)
            cp1.wait()
            cp2.wait()

            def ebody(e, cc):
                dd1 = da_v[e]
                dd2 = db_v[e]
                for g in range(C // 16):
                    a1 = i1_v[e, pl.ds(g * 16, 16)]
                    a3 = i1_v[e, pl.ds(2 * C + g * 16, 16)]
                    row_v[e, pl.ds(g * 16, 16)] = pa_v[e, pl.ds(g * 16, 16)] * a3 + dd1 * a1
                    row_v[e, pl.ds(C + g * 16, 16)] = pb_v[e, pl.ds(g * 16, 16)] * a3 + dd2 * a1
                return cc

            lax.fori_loop(0, K, ebody, 0)

        pltpu.sync_copy(row_v, acc.at[idxd_v], add=True)
        return carry

    lax.fori_loop(0, EPT // K, chunk, 0)
    plsc.subcore_barrier()

    for q in range(NPT // NPQ):
        pltpu.sync_copy(acc.at[pl.ds(nbase + q * NPQ, NPQ)], cb_v)
        pltpu.sync_copy(cb_v, out_hbm.at[c, pl.ds(nbase + q * NPQ, NPQ)])


def kernel(p1, p3, basis, diff, ind_2, W_pp, b_pp, W_pi, b_pi, W_ii):
    f32 = jnp.float32
    W_pi_r = W_pi.reshape(D_IN, C, NB).transpose(0, 2, 1).reshape(D_IN, C * NB)
    b_pi_r = b_pi.reshape(C, NB).T.reshape(1, C * NB)
    b_pp_r = b_pp.reshape(1, C)

    idx_flat = ind_2.reshape(2 * E)
    idx_i = ind_2[:, 0]
    idx_j = ind_2[:, 1]
    d0 = diff[:, 0]
    d1 = diff[:, 1]
    d2 = diff[:, 2]
    p3t = p3.transpose(1, 0, 2)

    nblk = 2000
    h = pl.pallas_call(
        _node_ff_body,
        grid=(N // nblk,),
        in_specs=[
            pl.BlockSpec((nblk, D_IN), lambda i: (i, 0)),
            pl.BlockSpec((D_IN, C), lambda i: (0, 0)),
            pl.BlockSpec((1, C), lambda i: (0, 0)),
        ],
        out_specs=pl.BlockSpec((nblk, C), lambda i: (i, 0)),
        out_shape=jax.ShapeDtypeStruct((N, C), f32),
    )(p1, W_pp, b_pp_r)

    hcat = _sc_gather(h, idx_flat).reshape(E, 2 * C)

    eblk = 512
    i1cat = pl.pallas_call(
        _edge_ff_body,
        grid=(E // eblk,),
        in_specs=[
            pl.BlockSpec((eblk, 2 * C), lambda i: (i, 0)),
            pl.BlockSpec((eblk, NB), lambda i: (i, 0)),
            pl.BlockSpec((2 * C, C * NB), lambda i: (0, 0)),
            pl.BlockSpec((1, C * NB), lambda i: (0, 0)),
            pl.BlockSpec((C, 3 * C), lambda i: (0, 0)),
        ],
        out_specs=pl.BlockSpec((eblk, 3 * C), lambda i: (i, 0)),
        out_shape=jax.ShapeDtypeStruct((E, 3 * C), f32),
    )(hcat, basis, W_pi_r, b_pi_r, W_ii)

    out01 = _sc_scatter(i1cat, idx_i, idx_j, d0, d1, d2,
                        p3t[0], p3t[1], p3t[2])

    p1t1, p3t1 = pl.pallas_call(
        _finalize_body,
        grid=(N // nblk,),
        in_specs=[
            pl.BlockSpec((nblk, 2 * C), lambda i: (i, 0)),
            pl.BlockSpec((nblk, 2 * C), lambda i: (i, 0)),
        ],
        out_specs=[
            pl.BlockSpec((nblk, C), lambda i: (i, 0)),
            pl.BlockSpec((nblk, 3, C), lambda i: (i, 0, 0)),
        ],
        out_shape=[
            jax.ShapeDtypeStruct((N, C), f32),
            jax.ShapeDtypeStruct((N, 3, C), f32),
        ],
    )(out01[0], out01[1])

    return (p1t1, p3t1)


# speedup vs baseline: 13.6934x; 13.6934x over previous
import functools

import jax
import jax.numpy as jnp
from jax import lax
from jax.experimental import pallas as pl
from jax.experimental.pallas import tpu as pltpu
from jax.experimental.pallas import tpu_sc as plsc

N = 10000
E = 320000
D_IN = 128
C = 64
NB = 4

NC = 2
NS = 16
NW = NC * NS

K = 80
GPW = 2 * E // NW
G2PW = E // NW
EPT = E // NS
NPAD = 10240
NPT = NPAD // NS
NPQ = 128

_MESH = plsc.VectorSubcoreMesh(core_axis_name="c", subcore_axis_name="s")


def _node_ff_body(p1_ref, p3_ref, wpp_ref, bpp_ref, t1_ref, t2_ref):
    x = jnp.dot(p1_ref[...], wpp_ref[...], preferred_element_type=jnp.float32)
    t1_ref[:, 0:C] = jnp.maximum(x + bpp_ref[...], 0.0)
    t1_ref[:, C:2 * C] = p3_ref[:, 0, :]
    t2_ref[:, 0:C] = p3_ref[:, 1, :]
    t2_ref[:, C:2 * C] = p3_ref[:, 2, :]


def _edge_ff_body(ga_ref, gb_ref, basis_ref, diff_ref, wt_ref, wb_ref,
                  bpi_ref, wii_ref, s0_ref, s1_ref):
    ga = ga_ref[...]
    hi = ga[:, 0:C]
    hj = ga[:, 2 * C:3 * C]
    p30 = ga[:, 3 * C:4 * C]
    gb = gb_ref[...]
    p31 = gb[:, 0:C]
    p32 = gb[:, C:2 * C]
    inter = (jnp.dot(hi, wt_ref[...], preferred_element_type=jnp.float32)
             + jnp.dot(hj, wb_ref[...], preferred_element_type=jnp.float32)
             + bpi_ref[...])
    bs = basis_ref[...]
    i1 = (inter[:, 0:C] * bs[:, 0:1] + inter[:, C:2 * C] * bs[:, 1:2]
          + inter[:, 2 * C:3 * C] * bs[:, 2:3] + inter[:, 3 * C:4 * C] * bs[:, 3:4])
    t = jnp.maximum(jnp.dot(i1, wii_ref[...], preferred_element_type=jnp.float32), 0.0)
    i1_1 = t[:, 0:C]
    i1_2 = t[:, C:2 * C]
    i1_3 = t[:, 2 * C:3 * C]
    df = diff_ref[...]
    s0_ref[:, 0:C] = i1_2
    s0_ref[:, C:2 * C] = p30 * i1_3 + df[:, 0:1] * i1_1
    s1_ref[:, 0:C] = p31 * i1_3 + df[:, 1:2] * i1_1
    s1_ref[:, C:2 * C] = p32 * i1_3 + df[:, 2:3] * i1_1


def _finalize_body(a_ref, b_ref, p1t_ref, p3t_ref):
    a = a_ref[...]
    b = b_ref[...]
    p1n = a[:, 0:C]
    p30 = a[:, C:2 * C]
    p31 = b[:, 0:C]
    p32 = b[:, C:2 * C]
    s = p30 * p30 + p31 * p31 + p32 * p32 + p1n
    p1t_ref[...] = s
    p3t_ref[:, 0, :] = p30 * s
    p3t_ref[:, 1, :] = p31 * s
    p3t_ref[:, 2, :] = p32 * s


@functools.partial(
    pl.kernel,
    out_type=(jax.ShapeDtypeStruct((2 * E, 2 * C), jnp.float32),
              jax.ShapeDtypeStruct((E, 2 * C), jnp.float32)),
    mesh=_MESH,
    scratch_types=[
        pltpu.VMEM((K,), jnp.int32),
        pltpu.VMEM((K, 2 * C), jnp.float32),
        pltpu.SemaphoreType.DMA,
    ],
)
def _sc_gather(t1_hbm, t2_hbm, idx2_hbm, idxj_hbm, ga_hbm, gb_hbm,
               idx_v, rows_v, sem):
    wid = lax.axis_index("s") * NC + lax.axis_index("c")

    basea = wid * GPW

    def chunk_a(t, carry):
        base = pl.multiple_of(basea + t * K, 8)
        pltpu.sync_copy(idx2_hbm.at[pl.ds(base, K)], idx_v)
        pltpu.async_copy(t1_hbm.at[idx_v], rows_v, sem).wait()
        pltpu.sync_copy(rows_v, ga_hbm.at[pl.ds(base, K)])
        return carry

    lax.fori_loop(0, GPW // K, chunk_a, 0)

    baseb = wid * G2PW

    def chunk_b(t, carry):
        base = pl.multiple_of(baseb + t * K, 8)
        pltpu.sync_copy(idxj_hbm.at[pl.ds(base, K)], idx_v)
        pltpu.async_copy(t2_hbm.at[idx_v], rows_v, sem).wait()
        pltpu.sync_copy(rows_v, gb_hbm.at[pl.ds(base, K)])
        return carry

    lax.fori_loop(0, G2PW // K, chunk_b, 0)


@functools.partial(
    pl.kernel,
    out_type=jax.ShapeDtypeStruct((NC, NPAD, 2 * C), jnp.float32),
    mesh=_MESH,
    scratch_types=[
        pltpu.VMEM((K,), jnp.int32),
        pltpu.VMEM((K, 2 * C), jnp.float32),
        pltpu.VMEM((NPQ, 2 * C), jnp.float32),
        pltpu.VMEM_SHARED((NPAD, 2 * C), jnp.float32),
    ],
)
def _sc_scatter(s0_hbm, s1_hbm, ii_hbm, out_hbm, idx_v, row_v, cb_v, acc):
    c = lax.axis_index("c")
    s = lax.axis_index("s")

    zero16 = jnp.zeros((16,), jnp.float32)

    def zrow(r, carry):
        for g in range(2 * C // 16):
            cb_v[r, pl.ds(g * 16, 16)] = zero16
        return carry

    lax.fori_loop(0, NPQ, zrow, 0)
    nbase = s * NPT
    for q in range(NPT // NPQ):
        pltpu.sync_copy(cb_v, acc.at[pl.ds(nbase + q * NPQ, NPQ)])
    plsc.subcore_barrier()

    ebase = s * EPT

    def chunk(t, carry):
        base = pl.multiple_of(ebase + t * K, 8)
        pltpu.sync_copy(ii_hbm.at[pl.ds(base, K)], idx_v)

        @pl.when(c == 0)
        def _():
            pltpu.sync_copy(s0_hbm.at[pl.ds(base, K)], row_v)

        @pl.when(c == 1)
        def _():
            pltpu.sync_copy(s1_hbm.at[pl.ds(base, K)], row_v)

        pltpu.sync_copy(row_v, acc.at[idx_v], add=True)
        return carry

    lax.fori_loop(0, EPT // K, chunk, 0)
    plsc.subcore_barrier()

    for q in range(NPT // NPQ):
        pltpu.sync_copy(acc.at[pl.ds(nbase + q * NPQ, NPQ)], cb_v)
        pltpu.sync_copy(cb_v, out_hbm.at[c, pl.ds(nbase + q * NPQ, NPQ)])


def kernel(p1, p3, basis, diff, ind_2, W_pp, b_pp, W_pi, b_pi, W_ii):
    f32 = jnp.float32
    W_pi_r = W_pi.reshape(D_IN, C, NB).transpose(0, 2, 1).reshape(D_IN, C * NB)
    W_t = W_pi_r[0:C]
    W_b = W_pi_r[C:2 * C]
    b_pi_r = b_pi.reshape(C, NB).T.reshape(1, C * NB)
    b_pp_r = b_pp.reshape(1, C)

    idx_flat = ind_2.reshape(2 * E)
    idx_i = ind_2[:, 0]
    idx_j = ind_2[:, 1]

    nblk = 2000
    table1, table2 = pl.pallas_call(
        _node_ff_body,
        grid=(N // nblk,),
        in_specs=[
            pl.BlockSpec((nblk, D_IN), lambda i: (i, 0)),
            pl.BlockSpec((nblk, 3, C), lambda i: (i, 0, 0)),
            pl.BlockSpec((D_IN, C), lambda i: (0, 0)),
            pl.BlockSpec((1, C), lambda i: (0, 0)),
        ],
        out_specs=[
            pl.BlockSpec((nblk, 2 * C), lambda i: (i, 0)),
            pl.BlockSpec((nblk, 2 * C), lambda i: (i, 0)),
        ],
        out_shape=[
            jax.ShapeDtypeStruct((N, 2 * C), f32),
            jax.ShapeDtypeStruct((N, 2 * C), f32),
        ],
    )(p1, p3, W_pp, b_pp_r)

    ga, gb = _sc_gather(table1, table2, idx_flat, idx_j)
    ga = ga.reshape(E, 4 * C)

    eblk = 512
    s0, s1 = pl.pallas_call(
        _edge_ff_body,
        grid=(E // eblk,),
        in_specs=[
            pl.BlockSpec((eblk, 4 * C), lambda i: (i, 0)),
            pl.BlockSpec((eblk, 2 * C), lambda i: (i, 0)),
            pl.BlockSpec((eblk, NB), lambda i: (i, 0)),
            pl.BlockSpec((eblk, 3), lambda i: (i, 0)),
            pl.BlockSpec((C, C * NB), lambda i: (0, 0)),
            pl.BlockSpec((C, C * NB), lambda i: (0, 0)),
            pl.BlockSpec((1, C * NB), lambda i: (0, 0)),
            pl.BlockSpec((C, 3 * C), lambda i: (0, 0)),
        ],
        out_specs=[
            pl.BlockSpec((eblk, 2 * C), lambda i: (i, 0)),
            pl.BlockSpec((eblk, 2 * C), lambda i: (i, 0)),
        ],
        out_shape=[
            jax.ShapeDtypeStruct((E, 2 * C), f32),
            jax.ShapeDtypeStruct((E, 2 * C), f32),
        ],
    )(ga, gb, basis, diff, W_t, W_b, b_pi_r, W_ii)

    out01 = _sc_scatter(s0, s1, idx_i)[:, :N]

    p1t1, p3t1 = pl.pallas_call(
        _finalize_body,
        grid=(N // nblk,),
        in_specs=[
            pl.BlockSpec((nblk, 2 * C), lambda i: (i, 0)),
            pl.BlockSpec((nblk, 2 * C), lambda i: (i, 0)),
        ],
        out_specs=[
            pl.BlockSpec((nblk, C), lambda i: (i, 0)),
            pl.BlockSpec((nblk, 3, C), lambda i: (i, 0, 0)),
        ],
        out_shape=[
            jax.ShapeDtypeStruct((N, C), f32),
            jax.ShapeDtypeStruct((N, 3, C), f32),
        ],
    )(out01[0], out01[1])

    return (p1t1, p3t1)


# speedup vs baseline: 23.7408x; 1.7337x over previous
import functools

import jax
import jax.numpy as jnp
from jax import lax
from jax.experimental import pallas as pl
from jax.experimental.pallas import tpu as pltpu
from jax.experimental.pallas import tpu_sc as plsc

N = 10000
E = 320000
D_IN = 128
C = 64
NB = 4

NC = 2
NS = 16
NW = NC * NS

K = 80
G = 5
GR = G * K
RPW = E // NW
NGRP = RPW // GR
EPT = E // NS
G2 = 2
GR2 = G2 * K
NGRPS = EPT // GR2
NPAD = 10240
NPT = NPAD // NS

_MESH = plsc.VectorSubcoreMesh(core_axis_name="c", subcore_axis_name="s")


def _node_ff_body(p1_ref, p3_ref, wpp_ref, bpp_ref, t1_ref, t2_ref):
    x = jnp.dot(p1_ref[...], wpp_ref[...], preferred_element_type=jnp.float32)
    t1_ref[:, 0:C] = jnp.maximum(x + bpp_ref[...], 0.0)
    t1_ref[:, C:2 * C] = p3_ref[:, 0, :]
    t2_ref[:, 0:C] = p3_ref[:, 1, :]
    t2_ref[:, C:2 * C] = p3_ref[:, 2, :]


def _edge_ff_body(gi_ref, gj1_ref, gj2_ref, basis_ref, diff_ref, wt_ref, wb_ref,
                  bpi_ref, wt6_ref, sel4_ref, seld_ref, s0_ref, s1_ref):
    bf = jnp.bfloat16
    gi = gi_ref[...]
    gj1 = gj1_ref[...]
    gj2 = gj2_ref[...]
    inter = (jnp.dot(gi.astype(bf), wt_ref[...], preferred_element_type=jnp.float32)
             + jnp.dot(gj1.astype(bf), wb_ref[...], preferred_element_type=jnp.float32)
             + bpi_ref[...])
    bse = jnp.dot(basis_ref[...], sel4_ref[...], preferred_element_type=jnp.float32)
    prod = (inter * bse).astype(bf)
    t6 = jnp.maximum(jnp.dot(prod, wt6_ref[...], preferred_element_type=jnp.float32), 0.0)
    dfe = jnp.dot(diff_ref[...], seld_ref[...], preferred_element_type=jnp.float32)
    a = t6[:, 0:2 * C]
    b = t6[:, 2 * C:4 * C]
    f = t6[:, 4 * C:6 * C]
    lane = lax.broadcasted_iota(jnp.int32, a.shape, 1)
    p = jnp.where(lane < C, 1.0, gj1)
    s0_ref[...] = a * p + dfe[:, 0:2 * C] * f
    s1_ref[...] = b * gj2 + dfe[:, 2 * C:4 * C] * f


def _finalize_body(a_ref, b_ref, p1t_ref, p3t_ref):
    a = a_ref[...]
    b = b_ref[...]
    p1n = a[:, 0:C]
    p30 = a[:, C:2 * C]
    p31 = b[:, 0:C]
    p32 = b[:, C:2 * C]
    s = p30 * p30 + p31 * p31 + p32 * p32 + p1n
    p1t_ref[...] = s
    p3t_ref[:, 0, :] = p30 * s
    p3t_ref[:, 1, :] = p31 * s
    p3t_ref[:, 2, :] = p32 * s


@functools.partial(
    pl.kernel,
    out_type=(jax.ShapeDtypeStruct((E, 2 * C), jnp.float32),
              jax.ShapeDtypeStruct((E, 2 * C), jnp.float32),
              jax.ShapeDtypeStruct((E, 2 * C), jnp.float32)),
    mesh=_MESH,
    scratch_types=[
        pltpu.VMEM((RPW,), jnp.int32),
        pltpu.VMEM((GR, 2 * C), jnp.float32),
        pltpu.VMEM((GR, 2 * C), jnp.float32),
        pltpu.SemaphoreType.DMA,
        pltpu.SemaphoreType.DMA,
        pltpu.SemaphoreType.DMA,
        pltpu.SemaphoreType.DMA,
    ],
)
def _sc_gather(t1_hbm, t2_hbm, ii_hbm, jj_hbm, gi_hbm, gj1_hbm, gj2_hbm,
               idx_v, rh0, rh1, g0, g1, o0, o1):
    wid = lax.axis_index("s") * NC + lax.axis_index("c")
    rbase = wid * RPW
    rh = (rh0, rh1)
    gsem = (g0, g1)
    osem = (o0, o1)

    def run_segment(src_idx_hbm, tab_hbm, out_hbm):
        pltpu.sync_copy(src_idx_hbm.at[pl.ds(pl.multiple_of(rbase, 8), RPW)], idx_v)

        def start_gathers(g, h):
            for b in range(G):
                cs = pl.multiple_of(g * GR + b * K, 8)
                pltpu.async_copy(tab_hbm.at[idx_v.at[pl.ds(cs, K)]],
                                 rh[h].at[pl.ds(b * K, K)], gsem[h])

        def wait_gathers(h):
            for b in range(G):
                pltpu.make_async_copy(tab_hbm.at[idx_v.at[pl.ds(0, K)]],
                                      rh[h].at[pl.ds(b * K, K)], gsem[h]).wait()

        def start_write(g, h):
            base = pl.multiple_of(rbase + g * GR, 8)
            pltpu.async_copy(rh[h], out_hbm.at[pl.ds(base, GR)], osem[h])

        def wait_write(h):
            pltpu.make_async_copy(rh[h], out_hbm.at[pl.ds(0, GR)], osem[h]).wait()

        start_gathers(0, 0)

        def body(t2, carry):
            for hh in range(2):
                g = 1 + 2 * t2 + hh
                h = (1 + hh) % 2

                @pl.when(g >= 2)
                def _():
                    wait_write(h)

                start_gathers(g, h)
                hp = 1 - h
                wait_gathers(hp)
                start_write(g - 1, hp)
            return carry

        lax.fori_loop(0, (NGRP - 1) // 2, body, 0)
        wait_gathers(0)
        start_write(NGRP - 1, 0)
        wait_write(1)
        wait_write(0)

    run_segment(ii_hbm, t1_hbm, gi_hbm)
    run_segment(jj_hbm, t1_hbm, gj1_hbm)
    run_segment(jj_hbm, t2_hbm, gj2_hbm)


@functools.partial(
    pl.kernel,
    out_type=jax.ShapeDtypeStruct((NC, NPAD, 2 * C), jnp.float32),
    mesh=_MESH,
    scratch_types=[
        pltpu.VMEM((GR2, 2 * C), jnp.float32),
        pltpu.VMEM((GR2, 2 * C), jnp.float32),
        [pltpu.VMEM((K,), jnp.int32) for _ in range(2 * G2)],
        pltpu.SemaphoreType.DMA,
        pltpu.SemaphoreType.DMA,
        pltpu.SemaphoreType.DMA,
        pltpu.SemaphoreType.DMA,
        pltpu.VMEM_SHARED((NPAD, 2 * C), jnp.float32),
    ],
)
def _sc_scatter(s0_hbm, s1_hbm, ii_hbm, out_hbm,
                rh0, rh1, ibs, g0, g1, o0, o1, acc):
    c = lax.axis_index("c")
    s = lax.axis_index("s")
    rh = (rh0, rh1)
    ib = (ibs[0:G2], ibs[G2:2 * G2])
    gsem = (g0, g1)
    osem = (o0, o1)

    zero16 = jnp.zeros((16,), jnp.float32)

    def zrow(r, carry):
        for q in range(2 * C // 16):
            rh0[r, pl.ds(q * 16, 16)] = zero16
        return carry

    lax.fori_loop(0, GR2, zrow, 0)
    nbase = s * NPT
    for q in range(NPT // GR2):
        pltpu.sync_copy(rh0, acc.at[pl.ds(nbase + q * GR2, GR2)])
    plsc.subcore_barrier()

    ebase = s * EPT

    def start_read(g, h):
        base = pl.multiple_of(ebase + g * GR2, 8)

        @pl.when(c == 0)
        def _():
            pltpu.async_copy(s0_hbm.at[pl.ds(base, GR2)], rh[h], gsem[h])

        @pl.when(c == 1)
        def _():
            pltpu.async_copy(s1_hbm.at[pl.ds(base, GR2)], rh[h], gsem[h])

        for b in range(G2):
            bb = pl.multiple_of(base + b * K, 8)
            pltpu.async_copy(ii_hbm.at[pl.ds(bb, K)], ib[h][b], gsem[h])

    def wait_read(h):
        pltpu.make_async_copy(s0_hbm.at[pl.ds(0, GR2)], rh[h], gsem[h]).wait()
        for b in range(G2):
            pltpu.make_async_copy(ii_hbm.at[pl.ds(0, K)], ib[h][b], gsem[h]).wait()

    def start_adds(h):
        for b in range(G2):
            pltpu.async_copy(rh[h].at[pl.ds(b * K, K)], acc.at[ib[h][b]],
                             osem[h], add=True)

    def wait_adds(h):
        for b in range(G2):
            pltpu.make_async_copy(rh[h].at[pl.ds(b * K, K)], acc.at[ib[h][0]],
                                  osem[h]).wait()

    start_read(0, 0)

    def body(t2, carry):
        for hh in range(2):
            g = 1 + 2 * t2 + hh
            h = (1 + hh) % 2

            @pl.when(g >= 2)
            def _():
                wait_adds(h)

            start_read(g, h)
            hp = 1 - h
            wait_read(hp)
            start_adds(hp)
        return carry

    lax.fori_loop(0, (NGRPS - 1) // 2, body, 0)
    wait_read(0)
    start_adds(0)
    wait_adds(1)
    wait_adds(0)
    plsc.subcore_barrier()

    for q in range(NPT // GR2):
        pltpu.sync_copy(acc.at[pl.ds(nbase + q * GR2, GR2)], rh0)
        pltpu.sync_copy(rh0, out_hbm.at[c, pl.ds(nbase + q * GR2, GR2)])


def kernel(p1, p3, basis, diff, ind_2, W_pp, b_pp, W_pi, b_pi, W_ii):
    f32 = jnp.float32
    bf = jnp.bfloat16
    W_pi_r = W_pi.reshape(D_IN, C, NB).transpose(0, 2, 1).reshape(D_IN, C * NB)
    zpad = jnp.zeros((C, C * NB), f32)
    W_t_pad = jnp.concatenate([W_pi_r[0:C], zpad], axis=0).astype(bf)
    W_b_pad = jnp.concatenate([W_pi_r[C:2 * C], zpad], axis=0).astype(bf)
    b_pi_r = b_pi.reshape(C, NB).T.reshape(1, C * NB)
    b_pp_r = b_pp.reshape(1, C)
    wii_1 = W_ii[:, 0:C]
    wii_2 = W_ii[:, C:2 * C]
    wii_3 = W_ii[:, 2 * C:3 * C]
    wt6 = jnp.tile(jnp.concatenate([wii_2, wii_3, wii_3, wii_3, wii_1, wii_1],
                                   axis=1), (NB, 1)).astype(bf)
    sel4 = jnp.repeat(jnp.eye(NB, dtype=f32), C, axis=1)
    seld = jnp.concatenate([jnp.zeros((3, C), f32),
                            jnp.repeat(jnp.eye(3, dtype=f32), C, axis=1)], axis=1)

    idx_flat = ind_2.reshape(2 * E)
    idx_i = idx_flat[0::2]
    idx_j = idx_flat[1::2]

    nblk = 2000
    table1, table2 = pl.pallas_call(
        _node_ff_body,
        grid=(N // nblk,),
        in_specs=[
            pl.BlockSpec((nblk, D_IN), lambda i: (i, 0)),
            pl.BlockSpec((nblk, 3, C), lambda i: (i, 0, 0)),
            pl.BlockSpec((D_IN, C), lambda i: (0, 0)),
            pl.BlockSpec((1, C), lambda i: (0, 0)),
        ],
        out_specs=[
            pl.BlockSpec((nblk, 2 * C), lambda i: (i, 0)),
            pl.BlockSpec((nblk, 2 * C), lambda i: (i, 0)),
        ],
        out_shape=[
            jax.ShapeDtypeStruct((N, 2 * C), f32),
            jax.ShapeDtypeStruct((N, 2 * C), f32),
        ],
    )(p1, p3, W_pp, b_pp_r)

    gi, gj1, gj2 = _sc_gather(table1, table2, idx_i, idx_j)

    eblk = 800
    s0, s1 = pl.pallas_call(
        _edge_ff_body,
        grid=(E // eblk,),
        in_specs=[
            pl.BlockSpec((eblk, 2 * C), lambda i: (i, 0)),
            pl.BlockSpec((eblk, 2 * C), lambda i: (i, 0)),
            pl.BlockSpec((eblk, 2 * C), lambda i: (i, 0)),
            pl.BlockSpec((eblk, NB), lambda i: (i, 0)),
            pl.BlockSpec((eblk, 3), lambda i: (i, 0)),
            pl.BlockSpec((D_IN, C * NB), lambda i: (0, 0)),
            pl.BlockSpec((D_IN, C * NB), lambda i: (0, 0)),
            pl.BlockSpec((1, C * NB), lambda i: (0, 0)),
            pl.BlockSpec((C * NB, 6 * C), lambda i: (0, 0)),
            pl.BlockSpec((NB, C * NB), lambda i: (0, 0)),
            pl.BlockSpec((3, C * NB), lambda i: (0, 0)),
        ],
        out_specs=[
            pl.BlockSpec((eblk, 2 * C), lambda i: (i, 0)),
            pl.BlockSpec((eblk, 2 * C), lambda i: (i, 0)),
        ],
        out_shape=[
            jax.ShapeDtypeStruct((E, 2 * C), f32),
            jax.ShapeDtypeStruct((E, 2 * C), f32),
        ],
    )(gi, gj1, gj2, basis, diff, W_t_pad, W_b_pad, b_pi_r, wt6, sel4, seld)

    out01 = _sc_scatter(s0, s1, idx_i)[:, :N]

    p1t1, p3t1 = pl.pallas_call(
        _finalize_body,
        grid=(N // nblk,),
        in_specs=[
            pl.BlockSpec((nblk, 2 * C), lambda i: (i, 0)),
            pl.BlockSpec((nblk, 2 * C), lambda i: (i, 0)),
        ],
        out_specs=[
            pl.BlockSpec((nblk, C), lambda i: (i, 0)),
            pl.BlockSpec((nblk, 3, C), lambda i: (i, 0, 0)),
        ],
        out_shape=[
            jax.ShapeDtypeStruct((N, C), f32),
            jax.ShapeDtypeStruct((N, 3, C), f32),
        ],
    )(out01[0], out01[1])

    return (p1t1, p3t1)


# speedup vs baseline: 27.7867x; 1.1704x over previous
import functools

import jax
import jax.numpy as jnp
from jax import lax
from jax.experimental import pallas as pl
from jax.experimental.pallas import tpu as pltpu
from jax.experimental.pallas import tpu_sc as plsc

N = 10000
E = 320000
D_IN = 128
C = 64
NB = 4

NC = 2
NS = 16
NW = NC * NS

K = 80
G = 5
GR = G * K
RPW = E // NW
NGRP = RPW // GR
EPT = E // NS
G2 = 2
GR2 = G2 * K
NGRPS = EPT // GR2
NPAD = 10240
NPT = NPAD // NS

_MESH = plsc.VectorSubcoreMesh(core_axis_name="c", subcore_axis_name="s")


def _pack2(lo, hi):
    lo16 = lax.bitcast_convert_type(lo.astype(jnp.bfloat16), jnp.uint16)
    hi16 = lax.bitcast_convert_type(hi.astype(jnp.bfloat16), jnp.uint16)
    w = hi16.astype(jnp.uint32) << 16 | lo16.astype(jnp.uint32)
    return lax.bitcast_convert_type(w, jnp.float32)


def _unpack2(w):
    u = lax.bitcast_convert_type(w, jnp.uint32)
    lo = lax.bitcast_convert_type((u & 0xFFFF).astype(jnp.uint16), jnp.bfloat16)
    hi = lax.bitcast_convert_type((u >> 16).astype(jnp.uint16), jnp.bfloat16)
    return lo, hi


def _node_ff_body(p1_ref, p3_ref, wpp_ref, bpp_ref, t1_ref):
    x = jnp.dot(p1_ref[...], wpp_ref[...], preferred_element_type=jnp.float32)
    h = jnp.maximum(x + bpp_ref[...], 0.0)
    t1_ref[:, 0:C] = _pack2(h, p3_ref[:, 1, :])
    t1_ref[:, C:2 * C] = _pack2(p3_ref[:, 0, :], p3_ref[:, 2, :])


def _edge_ff_body(gi_ref, gj_ref, basis_ref, diff_ref, wt_ref, wb_ref,
                  bpi_ref, wt6_ref, sel4_ref, seld_ref, s0_ref, s1_ref):
    bf = jnp.bfloat16
    lo_i, _ = _unpack2(gi_ref[...])
    lo_j, hi_j = _unpack2(gj_ref[...])
    inter = (jnp.dot(lo_i, wt_ref[...], preferred_element_type=jnp.float32)
             + jnp.dot(lo_j, wb_ref[...], preferred_element_type=jnp.float32)
             + bpi_ref[...])
    bse = jnp.dot(basis_ref[...], sel4_ref[...], preferred_element_type=jnp.float32)
    prod = (inter * bse).astype(bf)
    t6 = jnp.maximum(jnp.dot(prod, wt6_ref[...], preferred_element_type=jnp.float32), 0.0)
    dfe = jnp.dot(diff_ref[...], seld_ref[...], preferred_element_type=jnp.float32)
    a = t6[:, 0:2 * C]
    b = t6[:, 2 * C:4 * C]
    f = t6[:, 4 * C:6 * C]
    lane = lax.broadcasted_iota(jnp.int32, a.shape, 1)
    p = jnp.where(lane < C, jnp.float32(1.0), lo_j.astype(jnp.float32))
    s0_ref[...] = a * p + dfe[:, 0:2 * C] * f
    s1_ref[...] = b * hi_j.astype(jnp.float32) + dfe[:, 2 * C:4 * C] * f


def _finalize_body(a_ref, b_ref, p1t_ref, p3t_ref):
    a = a_ref[...]
    b = b_ref[...]
    p1n = a[:, 0:C]
    p30 = a[:, C:2 * C]
    p31 = b[:, 0:C]
    p32 = b[:, C:2 * C]
    s = p30 * p30 + p31 * p31 + p32 * p32 + p1n
    p1t_ref[...] = s
    p3t_ref[:, 0, :] = p30 * s
    p3t_ref[:, 1, :] = p31 * s
    p3t_ref[:, 2, :] = p32 * s


@functools.partial(
    pl.kernel,
    out_type=(jax.ShapeDtypeStruct((E, 2 * C), jnp.float32),
              jax.ShapeDtypeStruct((E, 2 * C), jnp.float32)),
    mesh=_MESH,
    compiler_params=pltpu.CompilerParams(needs_layout_passes=False),
    scratch_types=[
        pltpu.VMEM((2 * GR,), jnp.int32),
        pltpu.VMEM((2 * GR,), jnp.int32),
        pltpu.VMEM((GR,), jnp.int32),
        pltpu.VMEM((GR,), jnp.int32),
        pltpu.VMEM((GR, 2 * C), jnp.float32),
        pltpu.VMEM((GR, 2 * C), jnp.float32),
        pltpu.SemaphoreType.DMA,
        pltpu.SemaphoreType.DMA,
        pltpu.SemaphoreType.DMA,
        pltpu.SemaphoreType.DMA,
        pltpu.SemaphoreType.DMA,
        pltpu.SemaphoreType.DMA,
    ],
)
def _sc_gather(t1_hbm, idx2_hbm, gi_hbm, gj_hbm,
               pr0, pr1, ix0, ix1, rh0, rh1, p0, p1, g0, g1, o0, o1):
    wid = lax.axis_index("s") * NC + lax.axis_index("c")
    rbase = wid * RPW
    prs = (pr0, pr1)
    ixs = (ix0, ix1)
    rh = (rh0, rh1)
    psem = (p0, p1)
    gsem = (g0, g1)
    osem = (o0, o1)
    lanes = lax.iota(jnp.int32, 16)

    def run_segment(parity, out_hbm):

        def start_pairs(g, h):
            base = pl.multiple_of(2 * rbase + g * (2 * GR), 8)
            pltpu.async_copy(idx2_hbm.at[pl.ds(base, 2 * GR)],
                             prs[h], psem[h])

        def wait_pairs(h):
            pltpu.make_async_copy(idx2_hbm.at[pl.ds(0, 2 * GR)],
                                  prs[h], psem[h]).wait()

        def deint(h):
            for q in range(GR // 16):
                pos = lanes * 2 + (q * 32 + parity)
                ixs[h][pl.ds(q * 16, 16)] = plsc.load_gather(prs[h], [pos])

        def start_gathers(h):
            for b in range(G):
                pltpu.async_copy(
                    t1_hbm.at[ixs[h].at[pl.ds(b * K, K)]],
                    rh[h].at[pl.ds(b * K, K)], gsem[h])

        def wait_gathers(h):
            for b in range(G):
                pltpu.make_async_copy(
                    t1_hbm.at[ixs[h].at[pl.ds(0, K)]],
                    rh[h].at[pl.ds(b * K, K)], gsem[h]).wait()

        def start_write(g, h):
            base = pl.multiple_of(rbase + g * GR, 8)
            pltpu.async_copy(rh[h], out_hbm.at[pl.ds(base, GR)], osem[h])

        def wait_write(h):
            pltpu.make_async_copy(rh[h], out_hbm.at[pl.ds(0, GR)], osem[h]).wait()

        start_pairs(0, 0)
        wait_pairs(0)
        deint(0)
        start_gathers(0)
        start_pairs(1, 1)

        def body(t2, carry):
            for hh in range(2):
                g = 1 + 2 * t2 + hh
                h = (1 + hh) % 2

                @pl.when(g >= 2)
                def _():
                    wait_write(h)

                wait_pairs(h)
                deint(h)
                start_gathers(h)

                @pl.when(g + 1 <= NGRP - 1)
                def _():
                    start_pairs(g + 1, 1 - h)

                hp = 1 - h
                wait_gathers(hp)
                start_write(g - 1, hp)
            return carry

        lax.fori_loop(0, (NGRP - 1) // 2, body, 0)
        wait_gathers(0)
        start_write(NGRP - 1, 0)
        wait_write(1)
        wait_write(0)

    run_segment(0, gi_hbm)
    run_segment(1, gj_hbm)


@functools.partial(
    pl.kernel,
    out_type=jax.ShapeDtypeStruct((NC, NPAD, 2 * C), jnp.float32),
    mesh=_MESH,
    compiler_params=pltpu.CompilerParams(needs_layout_passes=False),
    scratch_types=[
        pltpu.VMEM((GR2, 2 * C), jnp.float32),
        pltpu.VMEM((GR2, 2 * C), jnp.float32),
        pltpu.VMEM((2 * GR2,), jnp.int32),
        pltpu.VMEM((2 * GR2,), jnp.int32),
        [pltpu.VMEM((K,), jnp.int32) for _ in range(2 * G2)],
        pltpu.SemaphoreType.DMA,
        pltpu.SemaphoreType.DMA,
        pltpu.SemaphoreType.DMA,
        pltpu.SemaphoreType.DMA,
        pltpu.VMEM_SHARED((NPAD, 2 * C), jnp.float32),
    ],
)
def _sc_scatter(s0_hbm, s1_hbm, idx2_hbm, out_hbm,
                rh0, rh1, pr0, pr1, ibs, g0, g1, o0, o1, acc):
    c = lax.axis_index("c")
    s = lax.axis_index("s")
    rh = (rh0, rh1)
    prs = (pr0, pr1)
    ib = (ibs[0:G2], ibs[G2:2 * G2])
    gsem = (g0, g1)
    osem = (o0, o1)
    lanes = lax.iota(jnp.int32, 16)

    zero16 = jnp.zeros((16,), jnp.float32)

    def zrow(r, carry):
        for q in range(2 * C // 16):
            rh0[r, pl.ds(q * 16, 16)] = zero16
        return carry

    lax.fori_loop(0, GR2, zrow, 0)
    nbase = s * NPT
    for q in range(NPT // GR2):
        pltpu.sync_copy(rh0, acc.at[pl.ds(nbase + q * GR2, GR2)])
    plsc.subcore_barrier()

    ebase = s * EPT

    def start_read(g, h):
        base = pl.multiple_of(ebase + g * GR2, 8)

        @pl.when(c == 0)
        def _():
            pltpu.async_copy(s0_hbm.at[pl.ds(base, GR2)], rh[h], gsem[h])

        @pl.when(c == 1)
        def _():
            pltpu.async_copy(s1_hbm.at[pl.ds(base, GR2)], rh[h], gsem[h])

        pltpu.async_copy(idx2_hbm.at[pl.ds(pl.multiple_of(2 * base, 8), 2 * GR2)],
                         prs[h], gsem[h])

    def wait_read(h):
        pltpu.make_async_copy(s0_hbm.at[pl.ds(0, GR2)], rh[h], gsem[h]).wait()
        pltpu.make_async_copy(idx2_hbm.at[pl.ds(0, 2 * GR2)],
                              prs[h], gsem[h]).wait()

    def deint(h):
        for b in range(G2):
            for q in range(K // 16):
                pos = lanes * 2 + 2 * (b * K + q * 16)
                ib[h][b][pl.ds(q * 16, 16)] = plsc.load_gather(prs[h], [pos])

    def start_adds(h):
        for b in range(G2):
            pltpu.async_copy(rh[h].at[pl.ds(b * K, K)],
                             acc.at[ib[h][b]], osem[h], add=True)

    def wait_adds(h):
        for b in range(G2):
            pltpu.make_async_copy(rh[h].at[pl.ds(b * K, K)],
                                  acc.at[ib[h][0]], osem[h]).wait()

    start_read(0, 0)

    def body(t2, carry):
        for hh in range(2):
            g = 1 + 2 * t2 + hh
            h = (1 + hh) % 2

            @pl.when(g >= 2)
            def _():
                wait_adds(h)

            start_read(g, h)
            hp = 1 - h
            wait_read(hp)
            deint(hp)
            start_adds(hp)
        return carry

    lax.fori_loop(0, (NGRPS - 1) // 2, body, 0)
    wait_read(0)
    deint(0)
    start_adds(0)
    wait_adds(1)
    wait_adds(0)
    plsc.subcore_barrier()

    for q in range(NPT // GR2):
        pltpu.sync_copy(acc.at[pl.ds(nbase + q * GR2, GR2)], rh0)
        pltpu.sync_copy(rh0, out_hbm.at[c, pl.ds(nbase + q * GR2, GR2)])


def kernel(p1, p3, basis, diff, ind_2, W_pp, b_pp, W_pi, b_pi, W_ii):
    f32 = jnp.float32
    bf = jnp.bfloat16
    W_pi_r = W_pi.reshape(D_IN, C, NB).transpose(0, 2, 1).reshape(D_IN, C * NB)
    zpad = jnp.zeros((C, C * NB), f32)
    W_t_pad = jnp.concatenate([W_pi_r[0:C], zpad], axis=0).astype(bf)
    W_b_pad = jnp.concatenate([W_pi_r[C:2 * C], zpad], axis=0).astype(bf)
    b_pi_r = b_pi.reshape(C, NB).T.reshape(1, C * NB)
    b_pp_r = b_pp.reshape(1, C)
    wii_1 = W_ii[:, 0:C]
    wii_2 = W_ii[:, C:2 * C]
    wii_3 = W_ii[:, 2 * C:3 * C]
    wt6 = jnp.tile(jnp.concatenate([wii_2, wii_3, wii_3, wii_3, wii_1, wii_1],
                                   axis=1), (NB, 1)).astype(bf)
    sel4 = jnp.repeat(jnp.eye(NB, dtype=f32), C, axis=1)
    seld = jnp.concatenate([jnp.zeros((3, C), f32),
                            jnp.repeat(jnp.eye(3, dtype=f32), C, axis=1)], axis=1)

    idx2 = ind_2.reshape(2 * E)

    nblk = 2000
    table1 = pl.pallas_call(
        _node_ff_body,
        grid=(N // nblk,),
        in_specs=[
            pl.BlockSpec((nblk, D_IN), lambda i: (i, 0)),
            pl.BlockSpec((nblk, 3, C), lambda i: (i, 0, 0)),
            pl.BlockSpec((D_IN, C), lambda i: (0, 0)),
            pl.BlockSpec((1, C), lambda i: (0, 0)),
        ],
        out_specs=pl.BlockSpec((nblk, 2 * C), lambda i: (i, 0)),
        out_shape=jax.ShapeDtypeStruct((N, 2 * C), f32),
    )(p1, p3, W_pp, b_pp_r)

    gi, gj = _sc_gather(table1, idx2)

    eblk = 800
    s0, s1 = pl.pallas_call(
        _edge_ff_body,
        grid=(E // eblk,),
        in_specs=[
            pl.BlockSpec((eblk, 2 * C), lambda i: (i, 0)),
            pl.BlockSpec((eblk, 2 * C), lambda i: (i, 0)),
            pl.BlockSpec((eblk, NB), lambda i: (i, 0)),
            pl.BlockSpec((eblk, 3), lambda i: (i, 0)),
            pl.BlockSpec((D_IN, C * NB), lambda i: (0, 0)),
            pl.BlockSpec((D_IN, C * NB), lambda i: (0, 0)),
            pl.BlockSpec((1, C * NB), lambda i: (0, 0)),
            pl.BlockSpec((C * NB, 6 * C), lambda i: (0, 0)),
            pl.BlockSpec((NB, C * NB), lambda i: (0, 0)),
            pl.BlockSpec((3, C * NB), lambda i: (0, 0)),
        ],
        out_specs=[
            pl.BlockSpec((eblk, 2 * C), lambda i: (i, 0)),
            pl.BlockSpec((eblk, 2 * C), lambda i: (i, 0)),
        ],
        out_shape=[
            jax.ShapeDtypeStruct((E, 2 * C), f32),
            jax.ShapeDtypeStruct((E, 2 * C), f32),
        ],
    )(gi, gj, basis, diff, W_t_pad, W_b_pad, b_pi_r, wt6, sel4, seld)

    out01 = _sc_scatter(s0, s1, idx2)[:, :N]

    p1t1, p3t1 = pl.pallas_call(
        _finalize_body,
        grid=(N // nblk,),
        in_specs=[
            pl.BlockSpec((nblk, 2 * C), lambda i: (i, 0)),
            pl.BlockSpec((nblk, 2 * C), lambda i: (i, 0)),
        ],
        out_specs=[
            pl.BlockSpec((nblk, C), lambda i: (i, 0)),
            pl.BlockSpec((nblk, 3, C), lambda i: (i, 0, 0)),
        ],
        out_shape=[
            jax.ShapeDtypeStruct((N, C), f32),
            jax.ShapeDtypeStruct((N, 3, C), f32),
        ],
    )(out01[0], out01[1])

    return (p1t1, p3t1)
